# baseline (speedup 1.0000x reference)
import functools

import jax
import jax.numpy as jnp
from jax import lax
from jax.experimental import pallas as pl
from jax.experimental.pallas import tpu as pltpu
from jax.experimental.pallas import tpu_sc as plsc

B = 16384
D = 32
NC = 2
NS = 16
NW = NC * NS
BPW = B // NW
CH = 128
NCH = BPW // CH

BLK = 2048


def _gather_body(ut, it, uid3, iid3, u_out, i_out, uidx, iidx, urows, irows, sem):
    wid = lax.axis_index("s") * NC + lax.axis_index("c")
    base = wid * BPW
    pltpu.sync_copy(uid3.at[wid], uidx)
    pltpu.sync_copy(iid3.at[wid], iidx)
    copies = []
    for j in range(NCH):
        copies.append(pltpu.async_copy(ut.at[uidx.at[j]], urows.at[j], sem))
        copies.append(pltpu.async_copy(it.at[iidx.at[j]], irows.at[j], sem))
    for c in copies:
        c.wait()
    for j in range(NCH):
        pltpu.sync_copy(urows.at[j], u_out.at[pl.ds(base + j * CH, CH)])
        pltpu.sync_copy(irows.at[j], i_out.at[pl.ds(base + j * CH, CH)])


def _sc_gather(user_table, item_table, uid3, iid3):
    mesh = plsc.VectorSubcoreMesh(core_axis_name="c", subcore_axis_name="s")
    fn = functools.partial(
        pl.kernel,
        mesh=mesh,
        out_type=(
            jax.ShapeDtypeStruct((B, D), jnp.float32),
            jax.ShapeDtypeStruct((B, D), jnp.float32),
        ),
        scratch_types=[
            pltpu.VMEM((NCH, CH), jnp.int32),
            pltpu.VMEM((NCH, CH), jnp.int32),
            pltpu.VMEM((NCH, CH, D), jnp.float32),
            pltpu.VMEM((NCH, CH, D), jnp.float32),
            pltpu.SemaphoreType.DMA,
        ],
        compiler_params=pltpu.CompilerParams(use_tc_tiling_on_sc=False),
    )(_gather_body)
    return fn(user_table, item_table, uid3, iid3)


def _mlp_body(u, i, w1, b1, w2, b2, w3t, b3, o):
    f = jnp.concatenate([u[...], i[...]], axis=1)
    h = jnp.maximum(
        jnp.dot(f, w1[...], preferred_element_type=jnp.float32) + b1[...], 0.0)
    h = jnp.maximum(
        jnp.dot(h, w2[...], preferred_element_type=jnp.float32) + b2[...], 0.0)
    z = jnp.sum(h * w3t[...], axis=1) + b3[0, 0]
    o[...] = jax.nn.sigmoid(z)


def _tc_mlp(u_emb, i_emb, W1, b1, W2, b2, W3, b3):
    b1r = b1.reshape(1, -1)
    b2r = b2.reshape(1, -1)
    w3t = W3.reshape(1, -1)
    b3r = b3.reshape(1, 1)
    grid = (B // BLK,)
    return pl.pallas_call(
        _mlp_body,
        grid=grid,
        in_specs=[
            pl.BlockSpec((BLK, D), lambda idx: (idx, 0)),
            pl.BlockSpec((BLK, D), lambda idx: (idx, 0)),
            pl.BlockSpec(W1.shape, lambda idx: (0, 0)),
            pl.BlockSpec(b1r.shape, lambda idx: (0, 0)),
            pl.BlockSpec(W2.shape, lambda idx: (0, 0)),
            pl.BlockSpec(b2r.shape, lambda idx: (0, 0)),
            pl.BlockSpec(w3t.shape, lambda idx: (0, 0)),
            pl.BlockSpec(memory_space=pltpu.SMEM),
        ],
        out_specs=pl.BlockSpec((BLK,), lambda idx: (idx,)),
        out_shape=jax.ShapeDtypeStruct((B,), jnp.float32),
    )(u_emb, i_emb, W1, b1r, W2, b2r, w3t, b3r)


def kernel(user_table, item_table, W1, b1, W2, b2, W3, b3, user_ids, item_ids):
    uid3 = user_ids.astype(jnp.int32).reshape(NW, NCH, CH)
    iid3 = item_ids.astype(jnp.int32).reshape(NW, NCH, CH)
    u_emb, i_emb = _sc_gather(user_table, item_table, uid3, iid3)
    return _tc_mlp(u_emb, i_emb, W1, b1, W2, b2, W3, b3)
